# Initial kernel scaffold; baseline (speedup 1.0000x reference)
import jax
import jax.numpy as jnp
from jax.experimental import pallas as pl


def kernel(inputs, edge_index, emb, eps, W0, b0, W1, b1, W2, b2, R0, rb0, R1, rb1, R2, rb2, attn_W, attn_b):
    raise NotImplementedError("write your pallas kernel here")


# speedup vs baseline: 1.0053x; 1.0053x over previous
import functools

import jax
import jax.numpy as jnp
from jax import lax
from jax.experimental import pallas as pl
from jax.experimental.pallas import tpu as pltpu

N = 10000
E = 160000
D = 256
L = 3

ROW_BLK = 1000


def _gin_layer_body(scale_ref, h_ref, agg_ref, w_ref, b_ref, r_ref, rb_ref, out_ref):
    x = scale_ref[0, 0] * h_ref[...] + agg_ref[...]
    h1 = jnp.dot(x, w_ref[...], preferred_element_type=jnp.float32) + b_ref[...]
    hr = jnp.maximum(h1, 0.0)
    h2 = jnp.dot(hr, r_ref[...], preferred_element_type=jnp.float32) + rb_ref[...]
    out_ref[...] = hr + jnp.maximum(h2, 0.0)


def _gin_layer3_attn_body(scale_ref, h_ref, agg_ref, w_ref, b_ref, r_ref, rb_ref,
                          h1_ref, h2_ref, aw_ref, ab_ref, out_ref):
    x = scale_ref[0, 0] * h_ref[...] + agg_ref[...]
    t1 = jnp.dot(x, w_ref[...], preferred_element_type=jnp.float32) + b_ref[...]
    hr = jnp.maximum(t1, 0.0)
    t2 = jnp.dot(hr, r_ref[...], preferred_element_type=jnp.float32) + rb_ref[...]
    h3 = hr + jnp.maximum(t2, 0.0)
    h1 = h1_ref[...]
    h2 = h2_ref[...]
    aw = aw_ref[...]
    ab = ab_ref[0, 0]
    s1 = jnp.sum(h1 * aw, axis=1, keepdims=True) + ab
    s2 = jnp.sum(h2 * aw, axis=1, keepdims=True) + ab
    s3 = jnp.sum(h3 * aw, axis=1, keepdims=True) + ab
    m = jnp.maximum(jnp.maximum(s1, s2), s3)
    e1 = jnp.exp(s1 - m)
    e2 = jnp.exp(s2 - m)
    e3 = jnp.exp(s3 - m)
    denom = e1 + e2 + e3
    out_ref[...] = (e1 * h1 + e2 * h2 + e3 * h3) / denom


def _row_spec():
    return pl.BlockSpec((ROW_BLK, D), lambda i: (i, 0))


def _full_spec(shape):
    return pl.BlockSpec(shape, lambda i: tuple(0 for _ in shape))


def _smem_spec(shape):
    return pl.BlockSpec(shape, lambda i: tuple(0 for _ in shape),
                        memory_space=pltpu.SMEM)


def _gin_layer(scale, h, agg, w, b, r, rb):
    grid = (N // ROW_BLK,)
    return pl.pallas_call(
        _gin_layer_body,
        grid=grid,
        in_specs=[
            _smem_spec((1, 1)),
            _row_spec(), _row_spec(),
            _full_spec((D, D)), _full_spec((1, D)),
            _full_spec((D, D)), _full_spec((1, D)),
        ],
        out_specs=_row_spec(),
        out_shape=jax.ShapeDtypeStruct((N, D), jnp.float32),
    )(scale, h, agg, w, b.reshape(1, D), r, rb.reshape(1, D))


def _gin_layer3_attn(scale, h, agg, w, b, r, rb, h1, h2, aw, ab):
    grid = (N // ROW_BLK,)
    return pl.pallas_call(
        _gin_layer3_attn_body,
        grid=grid,
        in_specs=[
            _smem_spec((1, 1)),
            _row_spec(), _row_spec(),
            _full_spec((D, D)), _full_spec((1, D)),
            _full_spec((D, D)), _full_spec((1, D)),
            _row_spec(), _row_spec(),
            _full_spec((1, D)), _smem_spec((1, 1)),
        ],
        out_specs=_row_spec(),
        out_shape=jax.ShapeDtypeStruct((N, D), jnp.float32),
    )(scale, h, agg, w, b.reshape(1, D), r, rb.reshape(1, D),
      h1, h2, aw.reshape(1, D), ab.reshape(1, 1))


def kernel(inputs, edge_index, emb, eps,
           W0, b0, W1, b1, W2, b2,
           R0, rb0, R1, rb1, R2, rb2,
           attn_W, attn_b):
    src = edge_index[0]
    dst = edge_index[1]
    Ws = [(W0, b0), (W1, b1), (W2, b2)]
    Rs = [(R0, rb0), (R1, rb1), (R2, rb2)]

    h = emb
    hidden = []
    for i in range(L):
        msg = jnp.take(h, src, axis=0)
        agg = jax.ops.segment_sum(msg, dst, num_segments=N)
        scale = (1.0 + eps[i]).reshape(1, 1)
        if i < L - 1:
            h = _gin_layer(scale, h, agg, Ws[i][0], Ws[i][1], Rs[i][0], Rs[i][1])
            hidden.append(h)
        else:
            node_out = _gin_layer3_attn(scale, h, agg, Ws[i][0], Ws[i][1],
                                        Rs[i][0], Rs[i][1],
                                        hidden[0], hidden[1], attn_W, attn_b)

    flat = inputs.reshape(-1)
    out = jnp.take(node_out, flat, axis=0).reshape(inputs.shape + (D,))
    return out
